# baseline (speedup 1.0000x reference)
import functools

import numpy as np
import jax
import jax.numpy as jnp
from jax import lax
from jax.experimental import pallas as pl
from jax.experimental.pallas import tpu as pltpu
from jax.experimental.pallas import tpu_sc as plsc

H = 512
W = 1024
K = 15
N = H * W
GAMMA = 0.1
NORM_CONST = float(N)

NC = 2
NS = 16
NW = NC * NS
PPW = N // NW
CHUNK = 2048
NSUB = CHUNK // 128
NG = CHUNK // 16
NCH = PPW // CHUNK


def _dirs_np():
    v = (np.arange(H, dtype=np.float64) + 0.5) / H
    u = (np.arange(W, dtype=np.float64) + 0.5) / W
    lat = np.pi / 2.0 - v * np.pi
    lon = u * 2.0 * np.pi - np.pi
    x = np.cos(lat)[:, None] * np.sin(lon)[None, :]
    y = np.sin(lat)[:, None] * np.ones_like(lon)[None, :]
    z = np.cos(lat)[:, None] * np.cos(lon)[None, :]
    return np.stack([x, y, z], axis=0).astype(np.float32)


_DIRS = _dirs_np()


def _build_body(depth_ref, s2_ref, dirs_ref, out_ref):
    d = depth_ref[...]
    s = jnp.zeros_like(d)
    for c in range(3):
        cam = d * dirs_ref[c]
        s2c = s2_ref[c]
        out_ref[c] = cam
        out_ref[3 + c] = s2c
        s = s + cam * s2c
    out_ref[6] = s
    out_ref[7] = jnp.zeros_like(d)


def _build_table(depth, s2, dirs):
    HC = 64
    grid = H // HC
    return pl.pallas_call(
        _build_body,
        grid=(grid,),
        in_specs=[
            pl.BlockSpec((HC, W), lambda i: (i, 0)),
            pl.BlockSpec((3, HC, W), lambda i: (0, i, 0)),
            pl.BlockSpec((3, HC, W), lambda i: (0, i, 0)),
        ],
        out_specs=pl.BlockSpec((8, HC, W), lambda i: (0, i, 0)),
        out_shape=jax.ShapeDtypeStruct((8, H, W), jnp.float32),
    )(depth, s2, dirs)


def _rsqrt16(x):
    i = plsc.bitcast(x, jnp.int32)
    i = jnp.int32(0x5F3759DF) - (i >> 1)
    y = plsc.bitcast(i, jnp.float32)
    for _ in range(2):
        y = y * (1.5 - 0.5 * x * y * y)
    return y


def _sqrt16(x):
    return jnp.where(x > 0.0, x * _rsqrt16(x), 0.0)


def _build_aos_body(t_soa, aos_out, c0, c1, c2, c3, c4, c5, rows_b, sem):
    wid = lax.axis_index("s") * NC + lax.axis_index("c")
    lanes = lax.iota(jnp.int32, 16)
    cols6 = [jnp.full((16,), c, jnp.int32) for c in range(6)]
    bufs = (c0, c1, c2, c3, c4, c5)

    def chunk_body(ch, t):
        base = wid * PPW + ch * CHUNK
        rbase = wid * (PPW // 128) + ch * NSUB
        for c in range(6):
            pltpu.sync_copy(t_soa.at[c, pl.ds(rbase, NSUB)], bufs[c])

        def grp(i, t2):
            row = i >> 3
            col = (i & 7) * 16
            ridx = i * 16 + lanes
            for c in range(6):
                plsc.store_scatter(rows_b, [ridx, cols6[c]],
                                   bufs[c][row, pl.ds(col, 16)])
            return t2
        lax.fori_loop(0, NG, grp, 0)
        pltpu.sync_copy(rows_b, aos_out.at[pl.ds(base, CHUNK)])
        return t
    lax.fori_loop(0, NCH, chunk_body, 0)


def _sc_build_aos(t_soa3):
    mesh = plsc.VectorSubcoreMesh(core_axis_name="c", subcore_axis_name="s")
    f = pl.kernel(
        _build_aos_body,
        out_type=jax.ShapeDtypeStruct((N, 8), jnp.float32),
        mesh=mesh,
        compiler_params=pltpu.CompilerParams(
            use_tc_tiling_on_sc=False, needs_layout_passes=False),
        scratch_types=[
            pltpu.VMEM((NSUB, 128), jnp.float32),
            pltpu.VMEM((NSUB, 128), jnp.float32),
            pltpu.VMEM((NSUB, 128), jnp.float32),
            pltpu.VMEM((NSUB, 128), jnp.float32),
            pltpu.VMEM((NSUB, 128), jnp.float32),
            pltpu.VMEM((NSUB, 128), jnp.float32),
            pltpu.VMEM((CHUNK, 8), jnp.float32),
            pltpu.SemaphoreType.DMA,
        ],
    )
    return f(t_soa3)


def _sc_body(t_aos, t_soa, w_hbm, nb_hbm, out_hbm,
             idx0, idx1, w0, w1, s2x_b, s2y_b, s2z_b, sd_b,
             rows0, rows1, acc1_b, red_b, sem0, sem1):
    wid = lax.axis_index("s") * NC + lax.axis_index("c")
    lanes = lax.iota(jnp.int32, 16)
    cols = [jnp.full((16,), c, jnp.int32) for c in range(6)]
    idxs = (idx0, idx1)
    ws = (w0, w1)
    rows = (rows0, rows1)
    sems = (sem0, sem1)

    def chunk_body(ch, carry):
        sum1, sum2 = carry
        rbase = wid * (PPW // 128) + ch * NSUB
        pltpu.sync_copy(t_soa.at[3, pl.ds(rbase, NSUB)], s2x_b)
        pltpu.sync_copy(t_soa.at[4, pl.ds(rbase, NSUB)], s2y_b)
        pltpu.sync_copy(t_soa.at[5, pl.ds(rbase, NSUB)], s2z_b)
        pltpu.sync_copy(t_soa.at[6, pl.ds(rbase, NSUB)], sd_b)

        def fetch(k, p):
            pltpu.sync_copy(nb_hbm.at[k, pl.ds(rbase, NSUB)], idxs[p])
            pltpu.sync_copy(w_hbm.at[k, pl.ds(rbase, NSUB)], ws[p])

            def issue(j, t):
                pltpu.async_copy(t_aos.at[idxs[p].at[j]],
                                 rows[p].at[pl.ds(j * 128, 128)], sems[p])
                return t
            lax.fori_loop(0, NSUB, issue, 0)

        def compute(k, p, sum2_in):
            rows_b = rows[p]
            w_b = ws[p]
            pltpu.make_async_copy(
                t_aos.at[pl.ds(0, CHUNK)], rows_b, sems[p]).wait()

            def one(i, s2c):
                row = i >> 3
                col = (i & 7) * 16
                ridx = i * 16 + lanes
                gx = plsc.load_gather(rows_b, [ridx, cols[0]])
                gy = plsc.load_gather(rows_b, [ridx, cols[1]])
                gz = plsc.load_gather(rows_b, [ridx, cols[2]])
                hx = plsc.load_gather(rows_b, [ridx, cols[3]])
                hy = plsc.load_gather(rows_b, [ridx, cols[4]])
                hz = plsc.load_gather(rows_b, [ridx, cols[5]])
                s2x = s2x_b[row, pl.ds(col, 16)]
                s2y = s2y_b[row, pl.ds(col, 16)]
                s2z = s2z_b[row, pl.ds(col, 16)]
                sd = sd_b[row, pl.ds(col, 16)]
                wv = w_b[row, pl.ds(col, 16)]
                a = sd - (gx * s2x + gy * s2y + gz * s2z)
                aw = a * wv
                sq = aw * aw
                if k == 0:
                    acc1_b[pl.ds(i * 16, 16)] = sq
                else:
                    plsc.addupdate(acc1_b.at[pl.ds(i * 16, 16)], sq)
                dx = s2x - hx
                dy = s2y - hy
                dz = s2z - hz
                d2 = dx * dx + dy * dy + dz * dz
                return s2c + wv * _sqrt16(d2)

            def grp(i, s2c):
                s2c = one(i * 2, s2c)
                return one(i * 2 + 1, s2c)

            return lax.fori_loop(0, NG // 2, grp, sum2_in)

        fetch(0, 0)
        for k in range(K):
            if k + 1 < K:
                fetch(k + 1, (k + 1) % 2)
            sum2 = compute(k, k % 2, sum2)

        def sq1(i, s1):
            v = acc1_b[pl.ds(i * 16, 16)]
            return s1 + _sqrt16(v)
        sum1 = lax.fori_loop(0, NG, sq1, sum1)
        return sum1, sum2

    z = jnp.zeros((16,), jnp.float32)
    sum1, sum2 = lax.fori_loop(0, NCH, chunk_body, (z, z))
    red_b[0, pl.ds(0, 16)] = sum1
    red_b[0, pl.ds(16, 16)] = sum2

    def zr(j, t):
        red_b[0, pl.ds(32 + j * 16, 16)] = z
        return t
    lax.fori_loop(0, 6, zr, 0)
    pltpu.sync_copy(red_b, out_hbm.at[wid])


def _sc_partials(t_aos, t_soa, weights, nb3):
    mesh = plsc.VectorSubcoreMesh(core_axis_name="c", subcore_axis_name="s")
    f = pl.kernel(
        _sc_body,
        out_type=jax.ShapeDtypeStruct((NW, 1, 128), jnp.float32),
        mesh=mesh,
        compiler_params=pltpu.CompilerParams(
            use_tc_tiling_on_sc=False, needs_layout_passes=False),
        scratch_types=[
            pltpu.VMEM((NSUB, 128), jnp.int32),
            pltpu.VMEM((NSUB, 128), jnp.int32),
            pltpu.VMEM((NSUB, 128), jnp.float32),
            pltpu.VMEM((NSUB, 128), jnp.float32),
            pltpu.VMEM((NSUB, 128), jnp.float32),
            pltpu.VMEM((NSUB, 128), jnp.float32),
            pltpu.VMEM((NSUB, 128), jnp.float32),
            pltpu.VMEM((NSUB, 128), jnp.float32),
            pltpu.VMEM((CHUNK, 8), jnp.float32),
            pltpu.VMEM((CHUNK, 8), jnp.float32),
            pltpu.VMEM((CHUNK,), jnp.float32),
            pltpu.VMEM((1, 128), jnp.float32),
            pltpu.SemaphoreType.DMA,
            pltpu.SemaphoreType.DMA,
        ],
    )
    return f(t_aos, t_soa, weights, nb3)


def _final_body(p_ref, out_ref):
    x = p_ref[...]
    s1 = jnp.sum(x[:, 0, 0:16])
    s2 = jnp.sum(x[:, 0, 16:32])
    val = (s1 + GAMMA * s2) * (1.0 / NORM_CONST)
    out_ref[...] = jnp.broadcast_to(val, (1, 1))


def _final(partials):
    return pl.pallas_call(
        _final_body,
        out_shape=jax.ShapeDtypeStruct((1, 1), jnp.float32),
    )(partials)


@jax.jit
def kernel(sig1, sig2, weights, neighbours):
    depth = sig1.reshape(H, W)
    s2 = sig2.reshape(3, H, W)
    dirs = jnp.asarray(_DIRS)
    t_soa = _build_table(depth, s2, dirs)
    t_flat = t_soa.reshape(8, N)
    t_soa3 = t_flat.reshape(8, N // 128, 128)
    t_aos = _sc_build_aos(t_soa3)
    w3 = weights.reshape(K, N // 128, 128)
    nb3 = neighbours.astype(jnp.int32).reshape(K, N // 128, 128)
    partials = _sc_partials(t_aos, t_soa3, w3, nb3)
    return _final(partials).reshape(())


# speedup vs baseline: 1.0075x; 1.0075x over previous
import functools

import numpy as np
import jax
import jax.numpy as jnp
from jax import lax
from jax.experimental import pallas as pl
from jax.experimental.pallas import tpu as pltpu
from jax.experimental.pallas import tpu_sc as plsc

H = 512
W = 1024
K = 15
N = H * W
GAMMA = 0.1
NORM_CONST = float(N)

NC = 2
NS = 16
NW = NC * NS
PPW = N // NW
CHUNK = 2048
NSUB = CHUNK // 128
NG = CHUNK // 16
NCH = PPW // CHUNK


def _dirs_np():
    v = (np.arange(H, dtype=np.float64) + 0.5) / H
    u = (np.arange(W, dtype=np.float64) + 0.5) / W
    lat = np.pi / 2.0 - v * np.pi
    lon = u * 2.0 * np.pi - np.pi
    x = np.cos(lat)[:, None] * np.sin(lon)[None, :]
    y = np.sin(lat)[:, None] * np.ones_like(lon)[None, :]
    z = np.cos(lat)[:, None] * np.cos(lon)[None, :]
    return np.stack([x, y, z], axis=0).astype(np.float32)


_DIRS = _dirs_np()


def _build_body(depth_ref, s2_ref, dirs_ref, out_ref):
    d = depth_ref[...]
    s = jnp.zeros_like(d)
    for c in range(3):
        cam = d * dirs_ref[c]
        s2c = s2_ref[c]
        out_ref[c] = cam
        out_ref[3 + c] = s2c
        s = s + cam * s2c
    out_ref[6] = s
    out_ref[7] = jnp.zeros_like(d)


def _build_table(depth, s2, dirs):
    HC = 64
    grid = H // HC
    return pl.pallas_call(
        _build_body,
        grid=(grid,),
        in_specs=[
            pl.BlockSpec((HC, W), lambda i: (i, 0)),
            pl.BlockSpec((3, HC, W), lambda i: (0, i, 0)),
            pl.BlockSpec((3, HC, W), lambda i: (0, i, 0)),
        ],
        out_specs=pl.BlockSpec((8, HC, W), lambda i: (0, i, 0)),
        out_shape=jax.ShapeDtypeStruct((8, H, W), jnp.float32),
    )(depth, s2, dirs)


def _rsqrt16(x):
    i = plsc.bitcast(x, jnp.int32)
    i = jnp.int32(0x5F3759DF) - (i >> 1)
    y = plsc.bitcast(i, jnp.float32)
    for _ in range(3):
        y = y * (1.5 - 0.5 * x * y * y)
    return y


def _sqrt16(x):
    return jnp.where(x > 0.0, x * _rsqrt16(x), 0.0)


def _build_aos_body(t_soa, aos_out, c0, c1, c2, c3, c4, c5, rows_b, sem):
    wid = lax.axis_index("s") * NC + lax.axis_index("c")
    lanes = lax.iota(jnp.int32, 16)
    cols6 = [jnp.full((16,), c, jnp.int32) for c in range(6)]
    bufs = (c0, c1, c2, c3, c4, c5)

    def chunk_body(ch, t):
        base = wid * PPW + ch * CHUNK
        rbase = wid * (PPW // 128) + ch * NSUB
        for c in range(6):
            pltpu.sync_copy(t_soa.at[c, pl.ds(rbase, NSUB)], bufs[c])

        def grp(i, t2):
            row = i >> 3
            col = (i & 7) * 16
            ridx = i * 16 + lanes
            for c in range(6):
                plsc.store_scatter(rows_b, [ridx, cols6[c]],
                                   bufs[c][row, pl.ds(col, 16)])
            return t2
        lax.fori_loop(0, NG, grp, 0)
        pltpu.sync_copy(rows_b, aos_out.at[pl.ds(base, CHUNK)])
        return t
    lax.fori_loop(0, NCH, chunk_body, 0)


def _sc_build_aos(t_soa3):
    mesh = plsc.VectorSubcoreMesh(core_axis_name="c", subcore_axis_name="s")
    f = pl.kernel(
        _build_aos_body,
        out_type=jax.ShapeDtypeStruct((N, 8), jnp.float32),
        mesh=mesh,
        compiler_params=pltpu.CompilerParams(
            use_tc_tiling_on_sc=False, needs_layout_passes=False),
        scratch_types=[
            pltpu.VMEM((NSUB, 128), jnp.float32),
            pltpu.VMEM((NSUB, 128), jnp.float32),
            pltpu.VMEM((NSUB, 128), jnp.float32),
            pltpu.VMEM((NSUB, 128), jnp.float32),
            pltpu.VMEM((NSUB, 128), jnp.float32),
            pltpu.VMEM((NSUB, 128), jnp.float32),
            pltpu.VMEM((CHUNK, 8), jnp.float32),
            pltpu.SemaphoreType.DMA,
        ],
    )
    return f(t_soa3)


def _sc_body(t_aos, t_soa, w_hbm, nb_hbm, out_hbm,
             idx0, idx1, w0, w1, s2x_b, s2y_b, s2z_b, sd_b,
             rows0, rows1, acc1_b, red_b, sem0, sem1):
    wid = lax.axis_index("s") * NC + lax.axis_index("c")
    lanes = lax.iota(jnp.int32, 16)
    cols = [jnp.full((16,), c, jnp.int32) for c in range(6)]
    idxs = (idx0, idx1)
    ws = (w0, w1)
    rows = (rows0, rows1)
    sems = (sem0, sem1)

    def chunk_body(ch, carry):
        sum1, sum2 = carry
        rbase = wid * (PPW // 128) + ch * NSUB
        pltpu.sync_copy(t_soa.at[3, pl.ds(rbase, NSUB)], s2x_b)
        pltpu.sync_copy(t_soa.at[4, pl.ds(rbase, NSUB)], s2y_b)
        pltpu.sync_copy(t_soa.at[5, pl.ds(rbase, NSUB)], s2z_b)
        pltpu.sync_copy(t_soa.at[6, pl.ds(rbase, NSUB)], sd_b)

        def fetch(k, p):
            pltpu.sync_copy(nb_hbm.at[k, pl.ds(rbase // 2, NSUB // 2)], idxs[p])
            pltpu.sync_copy(w_hbm.at[k, pl.ds(rbase, NSUB)], ws[p])

            def issue(j, t):
                pltpu.async_copy(t_aos.at[idxs[p].at[j]],
                                 rows[p].at[pl.ds(j * 256, 256)], sems[p])
                return t
            lax.fori_loop(0, NSUB // 2, issue, 0)

        def compute(k, p, sum2_in):
            rows_b = rows[p]
            w_b = ws[p]
            pltpu.make_async_copy(
                t_aos.at[pl.ds(0, CHUNK)], rows_b, sems[p]).wait()

            def grp(i, s2c):
                row = i >> 3
                col = (i & 7) * 16
                ridx = i * 16 + lanes
                gx = plsc.load_gather(rows_b, [ridx, cols[0]])
                gy = plsc.load_gather(rows_b, [ridx, cols[1]])
                gz = plsc.load_gather(rows_b, [ridx, cols[2]])
                hx = plsc.load_gather(rows_b, [ridx, cols[3]])
                hy = plsc.load_gather(rows_b, [ridx, cols[4]])
                hz = plsc.load_gather(rows_b, [ridx, cols[5]])
                s2x = s2x_b[row, pl.ds(col, 16)]
                s2y = s2y_b[row, pl.ds(col, 16)]
                s2z = s2z_b[row, pl.ds(col, 16)]
                sd = sd_b[row, pl.ds(col, 16)]
                wv = w_b[row, pl.ds(col, 16)]
                a = sd - (gx * s2x + gy * s2y + gz * s2z)
                aw = a * wv
                sq = aw * aw
                if k == 0:
                    acc1_b[pl.ds(i * 16, 16)] = sq
                else:
                    plsc.addupdate(acc1_b.at[pl.ds(i * 16, 16)], sq)
                dx = s2x - hx
                dy = s2y - hy
                dz = s2z - hz
                d2 = dx * dx + dy * dy + dz * dz
                return s2c + wv * _sqrt16(d2)

            return lax.fori_loop(0, NG, grp, sum2_in)

        fetch(0, 0)
        for k in range(K):
            if k + 1 < K:
                fetch(k + 1, (k + 1) % 2)
            sum2 = compute(k, k % 2, sum2)

        def sq1(i, s1):
            v = acc1_b[pl.ds(i * 16, 16)]
            return s1 + _sqrt16(v)
        sum1 = lax.fori_loop(0, NG, sq1, sum1)
        return sum1, sum2

    z = jnp.zeros((16,), jnp.float32)
    sum1, sum2 = lax.fori_loop(0, NCH, chunk_body, (z, z))
    red_b[0, pl.ds(0, 16)] = sum1
    red_b[0, pl.ds(16, 16)] = sum2

    def zr(j, t):
        red_b[0, pl.ds(32 + j * 16, 16)] = z
        return t
    lax.fori_loop(0, 6, zr, 0)
    pltpu.sync_copy(red_b, out_hbm.at[wid])


def _sc_partials(t_aos, t_soa, weights, nb3):
    mesh = plsc.VectorSubcoreMesh(core_axis_name="c", subcore_axis_name="s")
    f = pl.kernel(
        _sc_body,
        out_type=jax.ShapeDtypeStruct((NW, 1, 128), jnp.float32),
        mesh=mesh,
        compiler_params=pltpu.CompilerParams(
            use_tc_tiling_on_sc=False, needs_layout_passes=False),
        scratch_types=[
            pltpu.VMEM((NSUB // 2, 256), jnp.int32),
            pltpu.VMEM((NSUB // 2, 256), jnp.int32),
            pltpu.VMEM((NSUB, 128), jnp.float32),
            pltpu.VMEM((NSUB, 128), jnp.float32),
            pltpu.VMEM((NSUB, 128), jnp.float32),
            pltpu.VMEM((NSUB, 128), jnp.float32),
            pltpu.VMEM((NSUB, 128), jnp.float32),
            pltpu.VMEM((NSUB, 128), jnp.float32),
            pltpu.VMEM((CHUNK, 8), jnp.float32),
            pltpu.VMEM((CHUNK, 8), jnp.float32),
            pltpu.VMEM((CHUNK,), jnp.float32),
            pltpu.VMEM((1, 128), jnp.float32),
            pltpu.SemaphoreType.DMA,
            pltpu.SemaphoreType.DMA,
        ],
    )
    return f(t_aos, t_soa, weights, nb3)


def _final_body(p_ref, out_ref):
    x = p_ref[...]
    s1 = jnp.sum(x[:, 0, 0:16])
    s2 = jnp.sum(x[:, 0, 16:32])
    val = (s1 + GAMMA * s2) * (1.0 / NORM_CONST)
    out_ref[...] = jnp.broadcast_to(val, (1, 1))


def _final(partials):
    return pl.pallas_call(
        _final_body,
        out_shape=jax.ShapeDtypeStruct((1, 1), jnp.float32),
    )(partials)


@jax.jit
def kernel(sig1, sig2, weights, neighbours):
    depth = sig1.reshape(H, W)
    s2 = sig2.reshape(3, H, W)
    dirs = jnp.asarray(_DIRS)
    t_soa = _build_table(depth, s2, dirs)
    t_flat = t_soa.reshape(8, N)
    t_soa3 = t_flat.reshape(8, N // 128, 128)
    t_aos = _sc_build_aos(t_soa3)
    w3 = weights.reshape(K, N // 128, 128)
    nb3 = neighbours.astype(jnp.int32).reshape(K, N // 256, 256)
    partials = _sc_partials(t_aos, t_soa3, w3, nb3)
    return _final(partials).reshape(())
